# Initial kernel scaffold; baseline (speedup 1.0000x reference)
import jax
import jax.numpy as jnp
from jax.experimental import pallas as pl


def kernel(token_ids, is_note, note_dur_idx, note_pitch_idx, other_idx, dur_w, pitch_w, other_w):
    raise NotImplementedError("write your pallas kernel here")


# speedup vs baseline: 7.6581x; 7.6581x over previous
import functools

import jax
import jax.numpy as jnp
from jax import lax
from jax.experimental import pallas as pl
from jax.experimental.pallas import tpu as pltpu
from jax.experimental.pallas import tpu_sc as plsc

D_MODEL = 1024
VOCAB = 128
N_DUR = 8
N_PITCH = 12
N_OTHER = 32

NUM_CORES = 2
NUM_SUBCORES = 16
NUM_WORKERS = NUM_CORES * NUM_SUBCORES
CHUNK = 32


def _table_body(mask_ref, nd_ref, np_ref, oi_ref, dur_ref, pitch_ref, oth_ref,
                comb_ref):
    nd = nd_ref[:]
    npi = np_ref[:]
    oi = oi_ref[:]
    oh_d = (nd == lax.broadcasted_iota(jnp.int32, (VOCAB, N_DUR), 1)
            ).astype(jnp.float32)
    oh_p = (npi == lax.broadcasted_iota(jnp.int32, (VOCAB, N_PITCH), 1)
            ).astype(jnp.float32)
    oh_o = (oi == lax.broadcasted_iota(jnp.int32, (VOCAB, N_OTHER), 1)
            ).astype(jnp.float32)
    note = (jnp.dot(oh_d, dur_ref[:], preferred_element_type=jnp.float32)
            + jnp.dot(oh_p, pitch_ref[:], preferred_element_type=jnp.float32))
    other = jnp.dot(oh_o, oth_ref[:], preferred_element_type=jnp.float32)
    mask = mask_ref[:]
    comb_ref[:] = mask * note + (1.0 - mask) * other


def _build_table(is_note, nd, npi, oi, dur_w, pitch_w, other_w):
    mask_f = is_note.astype(jnp.float32).reshape(VOCAB, 1)
    return pl.pallas_call(
        _table_body,
        out_shape=jax.ShapeDtypeStruct((VOCAB, D_MODEL), jnp.float32),
    )(mask_f, nd.reshape(VOCAB, 1), npi.reshape(VOCAB, 1),
      oi.reshape(VOCAB, 1), dur_w, pitch_w, other_w)


def _sc_body(n_tok, comb_hbm, tok_hbm, out_hbm, tid_v, rows_v,
             gsem0, gsem1, ssem0, ssem1):
    per_w = n_tok // NUM_WORKERS
    n_chunks = per_w // CHUNK
    wid = lax.axis_index("s") * NUM_CORES + lax.axis_index("c")
    base = wid * per_w
    pltpu.sync_copy(tok_hbm.at[pl.ds(base, per_w)], tid_v)

    def gdesc(g, buf, sem):
        return pltpu.make_async_copy(
            comb_hbm.at[tid_v.at[pl.ds(g * CHUNK, CHUNK)]],
            rows_v.at[buf], sem)

    def sdesc(g, buf, sem):
        return pltpu.make_async_copy(
            rows_v.at[buf], out_hbm.at[pl.ds(base + g * CHUNK, CHUNK)], sem)

    d0 = gdesc(0, 0, gsem0)
    d0.start()
    d1 = gdesc(1, 1, gsem1)
    d1.start()
    d0.wait()
    sdesc(0, 0, ssem0).start()
    d1.wait()
    sdesc(1, 1, ssem1).start()

    def body(k, carry):
        ga = 2 * k
        gb = ga + 1
        sdesc(ga, 0, ssem0).wait()
        e0 = gdesc(ga, 0, gsem0)
        e0.start()
        sdesc(gb, 1, ssem1).wait()
        e1 = gdesc(gb, 1, gsem1)
        e1.start()
        e0.wait()
        sdesc(ga, 0, ssem0).start()
        e1.wait()
        sdesc(gb, 1, ssem1).start()
        return carry

    lax.fori_loop(1, n_chunks // 2, body, 0)
    sdesc(n_chunks - 2, 0, ssem0).wait()
    sdesc(n_chunks - 1, 1, ssem1).wait()


def _sc_gather(comb, tok_flat):
    n_tok = tok_flat.shape[0]
    mesh = plsc.VectorSubcoreMesh(core_axis_name="c", subcore_axis_name="s")
    return pl.kernel(
        functools.partial(_sc_body, n_tok),
        out_type=jax.ShapeDtypeStruct((n_tok, D_MODEL), jnp.float32),
        mesh=mesh,
        scratch_types=[
            pltpu.VMEM((n_tok // NUM_WORKERS,), jnp.int32),
            pltpu.VMEM((2, CHUNK, D_MODEL), jnp.float32),
            pltpu.SemaphoreType.DMA,
            pltpu.SemaphoreType.DMA,
            pltpu.SemaphoreType.DMA,
            pltpu.SemaphoreType.DMA,
        ],
    )(comb, tok_flat)


def kernel(token_ids, is_note, note_dur_idx, note_pitch_idx, other_idx,
           dur_w, pitch_w, other_w):
    comb = _build_table(is_note, note_dur_idx, note_pitch_idx, other_idx,
                        dur_w, pitch_w, other_w)
    tok_flat = token_ids.reshape(-1)
    out_flat = _sc_gather(comb, tok_flat)
    return out_flat.reshape(token_ids.shape + (D_MODEL,))


# speedup vs baseline: 7.7620x; 1.0136x over previous
import functools

import jax
import jax.numpy as jnp
from jax import lax
from jax.experimental import pallas as pl
from jax.experimental.pallas import tpu as pltpu
from jax.experimental.pallas import tpu_sc as plsc

D_MODEL = 1024
VOCAB = 128
N_DUR = 8
N_PITCH = 12
N_OTHER = 32

NUM_CORES = 2
NUM_SUBCORES = 16
NUM_WORKERS = NUM_CORES * NUM_SUBCORES
CHUNK = 16
NBUF = 4


def _table_body(mask_ref, nd_ref, np_ref, oi_ref, dur_ref, pitch_ref, oth_ref,
                comb_ref):
    nd = nd_ref[:]
    npi = np_ref[:]
    oi = oi_ref[:]
    oh_d = (nd == lax.broadcasted_iota(jnp.int32, (VOCAB, N_DUR), 1)
            ).astype(jnp.float32)
    oh_p = (npi == lax.broadcasted_iota(jnp.int32, (VOCAB, N_PITCH), 1)
            ).astype(jnp.float32)
    oh_o = (oi == lax.broadcasted_iota(jnp.int32, (VOCAB, N_OTHER), 1)
            ).astype(jnp.float32)
    note = (jnp.dot(oh_d, dur_ref[:], preferred_element_type=jnp.float32)
            + jnp.dot(oh_p, pitch_ref[:], preferred_element_type=jnp.float32))
    other = jnp.dot(oh_o, oth_ref[:], preferred_element_type=jnp.float32)
    mask = mask_ref[:]
    comb_ref[:] = mask * note + (1.0 - mask) * other


def _build_table(is_note, nd, npi, oi, dur_w, pitch_w, other_w):
    mask_f = is_note.astype(jnp.float32).reshape(VOCAB, 1)
    return pl.pallas_call(
        _table_body,
        out_shape=jax.ShapeDtypeStruct((VOCAB, D_MODEL), jnp.float32),
    )(mask_f, nd.reshape(VOCAB, 1), npi.reshape(VOCAB, 1),
      oi.reshape(VOCAB, 1), dur_w, pitch_w, other_w)


def _sc_body(n_tok, comb_hbm, tok_hbm, out_hbm, tid_v, rows_v, *sems):
    gsems = sems[:NBUF]
    ssems = sems[NBUF:]
    per_w = n_tok // NUM_WORKERS
    n_chunks = per_w // CHUNK
    wid = lax.axis_index("s") * NUM_CORES + lax.axis_index("c")
    base = wid * per_w
    pltpu.sync_copy(tok_hbm.at[pl.ds(base, per_w)], tid_v)

    def gdesc(g, buf):
        return pltpu.make_async_copy(
            comb_hbm.at[tid_v.at[pl.ds(g * CHUNK, CHUNK)]],
            rows_v.at[buf], gsems[buf])

    def sdesc(g, buf):
        return pltpu.make_async_copy(
            rows_v.at[buf], out_hbm.at[pl.ds(base + g * CHUNK, CHUNK)],
            ssems[buf])

    for jj in range(NBUF):
        gdesc(jj, jj).start()
    gdesc(0, 0).wait()
    sdesc(0, 0).start()

    def body(k, carry):
        for jj in range(NBUF):
            i = NBUF * k + jj
            sdesc(i - NBUF, jj).wait()
            gdesc(i, jj).start()
            j = i - (NBUF - 1)
            b2 = (jj + 1) % NBUF
            gdesc(j, b2).wait()
            sdesc(j, b2).start()
        return carry

    lax.fori_loop(1, n_chunks // NBUF, body, 0)
    for jj in range(NBUF - 1):
        j = n_chunks - (NBUF - 1) + jj
        b2 = j % NBUF
        gdesc(j, b2).wait()
        sdesc(j, b2).start()
    for jj in range(NBUF):
        sdesc(n_chunks - NBUF + jj, jj).wait()


def _sc_gather(comb, tok_flat):
    n_tok = tok_flat.shape[0]
    mesh = plsc.VectorSubcoreMesh(core_axis_name="c", subcore_axis_name="s")
    return pl.kernel(
        functools.partial(_sc_body, n_tok),
        out_type=jax.ShapeDtypeStruct((n_tok, D_MODEL), jnp.float32),
        mesh=mesh,
        scratch_types=(
            [pltpu.VMEM((n_tok // NUM_WORKERS,), jnp.int32),
             pltpu.VMEM((NBUF, CHUNK, D_MODEL), jnp.float32)]
            + [pltpu.SemaphoreType.DMA] * (2 * NBUF)),
    )(comb, tok_flat)


def kernel(token_ids, is_note, note_dur_idx, note_pitch_idx, other_idx,
           dur_w, pitch_w, other_w):
    comb = _build_table(is_note, note_dur_idx, note_pitch_idx, other_idx,
                        dur_w, pitch_w, other_w)
    tok_flat = token_ids.reshape(-1)
    out_flat = _sc_gather(comb, tok_flat)
    return out_flat.reshape(token_ids.shape + (D_MODEL,))


# speedup vs baseline: 11.5182x; 1.4839x over previous
import functools

import jax
import jax.numpy as jnp
from jax import lax
from jax.experimental import pallas as pl
from jax.experimental.pallas import tpu as pltpu
from jax.experimental.pallas import tpu_sc as plsc

D_MODEL = 1024
VOCAB = 128
N_DUR = 8
N_PITCH = 12
N_OTHER = 32

NUM_CORES = 2
NUM_SUBCORES = 16
NUM_WORKERS = NUM_CORES * NUM_SUBCORES
CHUNK = 16
NBUF = 4
REPL = 32


def _table_body(mask_ref, nd_ref, np_ref, oi_ref, dur_ref, pitch_ref, oth_ref,
                comb_ref):
    nd = nd_ref[:]
    npi = np_ref[:]
    oi = oi_ref[:]
    oh_d = (nd == lax.broadcasted_iota(jnp.int32, (VOCAB, N_DUR), 1)
            ).astype(jnp.float32)
    oh_p = (npi == lax.broadcasted_iota(jnp.int32, (VOCAB, N_PITCH), 1)
            ).astype(jnp.float32)
    oh_o = (oi == lax.broadcasted_iota(jnp.int32, (VOCAB, N_OTHER), 1)
            ).astype(jnp.float32)
    note = (jnp.dot(oh_d, dur_ref[:], preferred_element_type=jnp.float32)
            + jnp.dot(oh_p, pitch_ref[:], preferred_element_type=jnp.float32))
    other = jnp.dot(oh_o, oth_ref[:], preferred_element_type=jnp.float32)
    mask = mask_ref[:]
    comb_ref[:] = mask * note + (1.0 - mask) * other


def _build_table(is_note, nd, npi, oi, dur_w, pitch_w, other_w):
    mask_f = is_note.astype(jnp.float32).reshape(VOCAB, 1)
    return pl.pallas_call(
        _table_body,
        grid=(REPL,),
        in_specs=[pl.BlockSpec((VOCAB, 1), lambda i: (0, 0))] * 4
        + [pl.BlockSpec((N_DUR, D_MODEL), lambda i: (0, 0)),
           pl.BlockSpec((N_PITCH, D_MODEL), lambda i: (0, 0)),
           pl.BlockSpec((N_OTHER, D_MODEL), lambda i: (0, 0))],
        out_specs=pl.BlockSpec((VOCAB, D_MODEL), lambda i: (i, 0)),
        out_shape=jax.ShapeDtypeStruct((REPL * VOCAB, D_MODEL), jnp.float32),
    )(mask_f, nd.reshape(VOCAB, 1), npi.reshape(VOCAB, 1),
      oi.reshape(VOCAB, 1), dur_w, pitch_w, other_w)


def _sc_body(n_tok, comb_hbm, tok_hbm, out_hbm, tid_v, rows_v, *sems):
    gsems = sems[:NBUF]
    ssems = sems[NBUF:]
    per_w = n_tok // NUM_WORKERS
    n_chunks = per_w // CHUNK
    wid = lax.axis_index("s") * NUM_CORES + lax.axis_index("c")
    base = wid * per_w
    pltpu.sync_copy(tok_hbm.at[pl.ds(base, per_w)], tid_v)

    def gdesc(g, buf):
        return pltpu.make_async_copy(
            comb_hbm.at[tid_v.at[pl.ds(g * CHUNK, CHUNK)]],
            rows_v.at[buf], gsems[buf])

    def sdesc(g, buf):
        return pltpu.make_async_copy(
            rows_v.at[buf], out_hbm.at[pl.ds(base + g * CHUNK, CHUNK)],
            ssems[buf])

    for jj in range(NBUF):
        gdesc(jj, jj).start()
    gdesc(0, 0).wait()
    sdesc(0, 0).start()

    def body(k, carry):
        for jj in range(NBUF):
            i = NBUF * k + jj
            sdesc(i - NBUF, jj).wait()
            gdesc(i, jj).start()
            j = i - (NBUF - 1)
            b2 = (jj + 1) % NBUF
            gdesc(j, b2).wait()
            sdesc(j, b2).start()
        return carry

    lax.fori_loop(1, n_chunks // NBUF, body, 0)
    for jj in range(NBUF - 1):
        j = n_chunks - (NBUF - 1) + jj
        b2 = j % NBUF
        gdesc(j, b2).wait()
        sdesc(j, b2).start()
    for jj in range(NBUF):
        sdesc(n_chunks - NBUF + jj, jj).wait()


def _sc_gather(comb, tok_flat):
    n_tok = tok_flat.shape[0]
    mesh = plsc.VectorSubcoreMesh(core_axis_name="c", subcore_axis_name="s")
    return pl.kernel(
        functools.partial(_sc_body, n_tok),
        out_type=jax.ShapeDtypeStruct((n_tok, D_MODEL), jnp.float32),
        mesh=mesh,
        scratch_types=(
            [pltpu.VMEM((n_tok // NUM_WORKERS,), jnp.int32),
             pltpu.VMEM((NBUF, CHUNK, D_MODEL), jnp.float32)]
            + [pltpu.SemaphoreType.DMA] * (2 * NBUF)),
    )(comb, tok_flat)


def kernel(token_ids, is_note, note_dur_idx, note_pitch_idx, other_idx,
           dur_w, pitch_w, other_w):
    comb = _build_table(is_note, note_dur_idx, note_pitch_idx, other_idx,
                        dur_w, pitch_w, other_w)
    tok_flat = token_ids.reshape(-1)
    n_tok = tok_flat.shape[0]
    per_w = n_tok // NUM_WORKERS
    rep_off = (jnp.arange(n_tok, dtype=jnp.int32) // per_w % REPL) * VOCAB
    out_flat = _sc_gather(comb, tok_flat + rep_off)
    return out_flat.reshape(token_ids.shape + (D_MODEL,))


# speedup vs baseline: 12.0067x; 1.0424x over previous
import functools

import jax
import jax.numpy as jnp
from jax import lax
from jax.experimental import pallas as pl
from jax.experimental.pallas import tpu as pltpu
from jax.experimental.pallas import tpu_sc as plsc

D_MODEL = 1024
VOCAB = 128
N_DUR = 8
N_PITCH = 12
N_OTHER = 32

NUM_CORES = 2
NUM_SUBCORES = 16
NUM_WORKERS = NUM_CORES * NUM_SUBCORES
CHUNK = 16
NBUF = 4
LAG = 2
REPL = 32


def _table_body(mask_ref, nd_ref, np_ref, oi_ref, dur_ref, pitch_ref, oth_ref,
                comb_ref, tab_v):
    @pl.when(pl.program_id(0) == 0)
    def _compute():
        nd = nd_ref[:]
        npi = np_ref[:]
        oi = oi_ref[:]
        oh_d = (nd == lax.broadcasted_iota(jnp.int32, (VOCAB, N_DUR), 1)
                ).astype(jnp.float32)
        oh_p = (npi == lax.broadcasted_iota(jnp.int32, (VOCAB, N_PITCH), 1)
                ).astype(jnp.float32)
        oh_o = (oi == lax.broadcasted_iota(jnp.int32, (VOCAB, N_OTHER), 1)
                ).astype(jnp.float32)
        note = (jnp.dot(oh_d, dur_ref[:], preferred_element_type=jnp.float32)
                + jnp.dot(oh_p, pitch_ref[:],
                          preferred_element_type=jnp.float32))
        other = jnp.dot(oh_o, oth_ref[:], preferred_element_type=jnp.float32)
        mask = mask_ref[:]
        tab_v[:] = mask * note + (1.0 - mask) * other

    comb_ref[:] = tab_v[:]


def _build_table(is_note, nd, npi, oi, dur_w, pitch_w, other_w):
    mask_f = is_note.astype(jnp.float32).reshape(VOCAB, 1)
    return pl.pallas_call(
        _table_body,
        grid=(REPL,),
        in_specs=[pl.BlockSpec((VOCAB, 1), lambda i: (0, 0))] * 4
        + [pl.BlockSpec((N_DUR, D_MODEL), lambda i: (0, 0)),
           pl.BlockSpec((N_PITCH, D_MODEL), lambda i: (0, 0)),
           pl.BlockSpec((N_OTHER, D_MODEL), lambda i: (0, 0))],
        out_specs=pl.BlockSpec((VOCAB, D_MODEL), lambda i: (i, 0)),
        out_shape=jax.ShapeDtypeStruct((REPL * VOCAB, D_MODEL), jnp.float32),
        scratch_shapes=[pltpu.VMEM((VOCAB, D_MODEL), jnp.float32)],
    )(mask_f, nd.reshape(VOCAB, 1), npi.reshape(VOCAB, 1),
      oi.reshape(VOCAB, 1), dur_w, pitch_w, other_w)


def _sc_body(n_tok, comb_hbm, tok_hbm, out_hbm, tid_v, rows_v, *sems):
    gsems = sems[:NBUF]
    ssems = sems[NBUF:]
    per_w = n_tok // NUM_WORKERS
    n_chunks = per_w // CHUNK
    wid = lax.axis_index("s") * NUM_CORES + lax.axis_index("c")
    base = wid * per_w
    pltpu.sync_copy(tok_hbm.at[pl.ds(base, per_w)], tid_v)

    def gdesc(g, buf):
        return pltpu.make_async_copy(
            comb_hbm.at[tid_v.at[pl.ds(g * CHUNK, CHUNK)]],
            rows_v.at[buf], gsems[buf])

    def sdesc(g, buf):
        return pltpu.make_async_copy(
            rows_v.at[buf], out_hbm.at[pl.ds(base + g * CHUNK, CHUNK)],
            ssems[buf])

    for jj in range(NBUF):
        gdesc(jj, jj).start()
    for jj in range(LAG, NBUF):
        j = jj - LAG
        gdesc(j, j % NBUF).wait()
        sdesc(j, j % NBUF).start()

    def body(k, carry):
        for jj in range(NBUF):
            i = NBUF * k + jj
            sdesc(i - NBUF, jj).wait()
            gdesc(i, jj).start()
            j = i - LAG
            b2 = (jj - LAG) % NBUF
            gdesc(j, b2).wait()
            sdesc(j, b2).start()
        return carry

    lax.fori_loop(1, n_chunks // NBUF, body, 0)
    for jj in range(LAG):
        j = n_chunks - LAG + jj
        b2 = j % NBUF
        gdesc(j, b2).wait()
        sdesc(j, b2).start()
    for jj in range(NBUF):
        sdesc(n_chunks - NBUF + jj, jj).wait()


def _sc_gather(comb, tok_flat):
    n_tok = tok_flat.shape[0]
    mesh = plsc.VectorSubcoreMesh(core_axis_name="c", subcore_axis_name="s")
    return pl.kernel(
        functools.partial(_sc_body, n_tok),
        out_type=jax.ShapeDtypeStruct((n_tok, D_MODEL), jnp.float32),
        mesh=mesh,
        scratch_types=(
            [pltpu.VMEM((n_tok // NUM_WORKERS,), jnp.int32),
             pltpu.VMEM((NBUF, CHUNK, D_MODEL), jnp.float32)]
            + [pltpu.SemaphoreType.DMA] * (2 * NBUF)),
    )(comb, tok_flat)


def kernel(token_ids, is_note, note_dur_idx, note_pitch_idx, other_idx,
           dur_w, pitch_w, other_w):
    comb = _build_table(is_note, note_dur_idx, note_pitch_idx, other_idx,
                        dur_w, pitch_w, other_w)
    tok_flat = token_ids.reshape(-1)
    n_tok = tok_flat.shape[0]
    per_w = n_tok // NUM_WORKERS
    rep_off = (jnp.arange(n_tok, dtype=jnp.int32) // per_w % REPL) * VOCAB
    out_flat = _sc_gather(comb, tok_flat + rep_off)
    return out_flat.reshape(token_ids.shape + (D_MODEL,))


# speedup vs baseline: 13.0448x; 1.0865x over previous
import functools

import jax
import jax.numpy as jnp
from jax import lax
from jax.experimental import pallas as pl
from jax.experimental.pallas import tpu as pltpu
from jax.experimental.pallas import tpu_sc as plsc

D_MODEL = 1024
VOCAB = 128
N_DUR = 8
N_PITCH = 12
N_OTHER = 32

NUM_CORES = 2
NUM_SUBCORES = 16
NUM_WORKERS = NUM_CORES * NUM_SUBCORES
CHUNK = 16
NBUF = 4
LAG = 2
REPL = 16


def _table_body(mask_ref, nd_ref, np_ref, oi_ref, dur_ref, pitch_ref, oth_ref,
                comb_ref, tab_v):
    @pl.when(pl.program_id(0) == 0)
    def _compute():
        mask = mask_ref[:]
        oh_d = (nd_ref[:] == lax.broadcasted_iota(
            jnp.int32, (N_DUR, VOCAB), 0)).astype(jnp.float32) * mask
        oh_p = (np_ref[:] == lax.broadcasted_iota(
            jnp.int32, (N_PITCH, VOCAB), 0)).astype(jnp.float32) * mask
        oh_o = (oi_ref[:] == lax.broadcasted_iota(
            jnp.int32, (N_OTHER, VOCAB), 0)).astype(jnp.float32) * (1.0 - mask)
        dims = (((0,), (0,)), ((), ()))
        tab_v[:] = (
            lax.dot_general(oh_d, dur_ref[:], dims,
                            preferred_element_type=jnp.float32)
            + lax.dot_general(oh_p, pitch_ref[:], dims,
                              preferred_element_type=jnp.float32)
            + lax.dot_general(oh_o, oth_ref[:], dims,
                              preferred_element_type=jnp.float32))

    comb_ref[:] = tab_v[:]


def _build_table(is_note, nd, npi, oi, dur_w, pitch_w, other_w):
    mask_f = is_note.astype(jnp.float32).reshape(1, VOCAB)
    return pl.pallas_call(
        _table_body,
        grid=(REPL,),
        in_specs=[pl.BlockSpec((1, VOCAB), lambda i: (0, 0))] * 4
        + [pl.BlockSpec((N_DUR, D_MODEL), lambda i: (0, 0)),
           pl.BlockSpec((N_PITCH, D_MODEL), lambda i: (0, 0)),
           pl.BlockSpec((N_OTHER, D_MODEL), lambda i: (0, 0))],
        out_specs=pl.BlockSpec((VOCAB, D_MODEL), lambda i: (i, 0)),
        out_shape=jax.ShapeDtypeStruct((REPL * VOCAB, D_MODEL), jnp.float32),
        scratch_shapes=[pltpu.VMEM((VOCAB, D_MODEL), jnp.float32)],
    )(mask_f, nd.reshape(1, VOCAB), npi.reshape(1, VOCAB),
      oi.reshape(1, VOCAB), dur_w, pitch_w, other_w)


def _sc_body(n_tok, comb_hbm, tok_hbm, out_hbm, tid_v, rows_v, *sems):
    gsems = sems[:NBUF]
    ssems = sems[NBUF:]
    per_w = n_tok // NUM_WORKERS
    n_chunks = per_w // CHUNK
    wid = lax.axis_index("s") * NUM_CORES + lax.axis_index("c")
    base = wid * per_w
    pltpu.sync_copy(tok_hbm.at[pl.ds(base, per_w)], tid_v)

    def gdesc(g, buf):
        return pltpu.make_async_copy(
            comb_hbm.at[tid_v.at[pl.ds(g * CHUNK, CHUNK)]],
            rows_v.at[buf], gsems[buf])

    def sdesc(g, buf):
        return pltpu.make_async_copy(
            rows_v.at[buf], out_hbm.at[pl.ds(base + g * CHUNK, CHUNK)],
            ssems[buf])

    for jj in range(NBUF):
        gdesc(jj, jj).start()
    for jj in range(LAG, NBUF):
        j = jj - LAG
        gdesc(j, j % NBUF).wait()
        sdesc(j, j % NBUF).start()

    def body(k, carry):
        for jj in range(NBUF):
            i = NBUF * k + jj
            sdesc(i - NBUF, jj).wait()
            gdesc(i, jj).start()
            j = i - LAG
            b2 = (jj - LAG) % NBUF
            gdesc(j, b2).wait()
            sdesc(j, b2).start()
        return carry

    lax.fori_loop(1, n_chunks // NBUF, body, 0)
    for jj in range(LAG):
        j = n_chunks - LAG + jj
        b2 = j % NBUF
        gdesc(j, b2).wait()
        sdesc(j, b2).start()
    for jj in range(NBUF):
        sdesc(n_chunks - NBUF + jj, jj).wait()


def _sc_gather(comb, tok_flat):
    n_tok = tok_flat.shape[0]
    mesh = plsc.VectorSubcoreMesh(core_axis_name="c", subcore_axis_name="s")
    return pl.kernel(
        functools.partial(_sc_body, n_tok),
        out_type=jax.ShapeDtypeStruct((n_tok, D_MODEL), jnp.float32),
        mesh=mesh,
        scratch_types=(
            [pltpu.VMEM((n_tok // NUM_WORKERS,), jnp.int32),
             pltpu.VMEM((NBUF, CHUNK, D_MODEL), jnp.float32)]
            + [pltpu.SemaphoreType.DMA] * (2 * NBUF)),
    )(comb, tok_flat)


def kernel(token_ids, is_note, note_dur_idx, note_pitch_idx, other_idx,
           dur_w, pitch_w, other_w):
    comb = _build_table(is_note, note_dur_idx, note_pitch_idx, other_idx,
                        dur_w, pitch_w, other_w)
    tok_flat = token_ids.reshape(-1)
    n_tok = tok_flat.shape[0]
    per_w = n_tok // NUM_WORKERS
    rep_off = (jnp.arange(n_tok, dtype=jnp.int32) // per_w % REPL) * VOCAB
    out_flat = _sc_gather(comb, tok_flat + rep_off)
    return out_flat.reshape(token_ids.shape + (D_MODEL,))
